# baseline (speedup 1.0000x reference)
import functools

import jax
import jax.numpy as jnp
from jax.experimental import pallas as pl
from jax.experimental.pallas import tpu as pltpu

L = 3
D = 128
H = 128
G = 128
C = 10

BLK = 2000


def _relu(x):
    return jnp.maximum(x, 0.0)


def _dot(a, b):
    return jax.lax.dot_general(a, b, (((1,), (0,)), ((), ())),
                               preferred_element_type=jnp.float32)


def _mlp_body(up_ref, b_ref, wu1, bu1, wu2, bu2, wb1, bb1, wb2, bb2,
              wc_u, wc_b, bc, out_ref):
    up = up_ref[...]
    bb = b_ref[...]
    hu = _relu(_dot(up, wu1[...]) + bu1[...])
    hu = _relu(_dot(hu, wu2[...]) + bu2[...])
    hb = _relu(_dot(bb, wb1[...]) + bb1[...])
    hb = _relu(_dot(hb, wb2[...]) + bb2[...])
    out_ref[...] = _relu(_dot(hu, wc_u[...]) + _dot(hb, wc_b[...]) + bc[...])


def _tc_mlp(out_up, out_b, p):
    n = out_up.shape[0]
    grid = n // BLK
    row_spec = pl.BlockSpec((BLK, H), lambda i: (i, 0))
    w_spec = pl.BlockSpec((H, H), lambda i: (0, 0))
    b_spec = pl.BlockSpec((1, H), lambda i: (0, 0))
    return pl.pallas_call(
        _mlp_body,
        grid=(grid,),
        in_specs=[row_spec, row_spec,
                  w_spec, b_spec, w_spec, b_spec,
                  w_spec, b_spec, w_spec, b_spec,
                  w_spec, w_spec, b_spec],
        out_specs=row_spec,
        out_shape=jax.ShapeDtypeStruct((n, H), jnp.float32),
    )(out_up, out_b,
      p["Wu1"], p["bu1"].reshape(1, H), p["Wu2"], p["bu2"].reshape(1, H),
      p["Wb1"], p["bb1"].reshape(1, H), p["Wb2"], p["bb2"].reshape(1, H),
      p["Wc"][:H], p["Wc"][H:], p["bc"].reshape(1, H))


def _pool_body(batch_ref, x_ref, out_ref, acc):
    i = pl.program_id(0)

    @pl.when(i == 0)
    def _():
        acc[...] = jnp.zeros_like(acc)

    b = batch_ref[0, 0, :]
    oh = (b[:, None] == jax.lax.broadcasted_iota(jnp.int32, (BLK, G), 1))
    oh = oh.astype(jnp.float32)
    acc[...] += jax.lax.dot_general(oh, x_ref[...], (((0,), (0,)), ((), ())),
                                    preferred_element_type=jnp.float32)

    @pl.when(i == pl.num_programs(0) - 1)
    def _():
        out_ref[...] = acc[...]


def _tc_pool(x, batch):
    n = x.shape[0]
    grid = n // BLK
    batch3 = batch.astype(jnp.int32).reshape(grid, 1, BLK)
    return pl.pallas_call(
        _pool_body,
        grid=(grid,),
        in_specs=[pl.BlockSpec((1, 1, BLK), lambda i: (i, 0, 0)),
                  pl.BlockSpec((BLK, H), lambda i: (i, 0))],
        out_specs=pl.BlockSpec((G, H), lambda i: (0, 0)),
        out_shape=jax.ShapeDtypeStruct((G, H), jnp.float32),
        scratch_shapes=[pltpu.VMEM((G, H), jnp.float32)],
    )(batch3, x)


def _readout_body(p0, p1, p2, w0, b0, w1, b1, w2, b2, w2f, b2f, out_ref):
    h = _relu(_dot(p0[...], w0[...]) + b0[...])
    h += _relu(_dot(p1[...], w1[...]) + b1[...])
    h += _relu(_dot(p2[...], w2[...]) + b2[...])
    out_ref[...] = _dot(h, w2f[...]) + b2f[...]


def _tc_readout(pooled, lin1, lin2w, lin2b):
    w2f = jnp.zeros((2 * H, 128), jnp.float32).at[:, :C].set(lin2w)
    b2f = jnp.zeros((1, 128), jnp.float32).at[0, :C].set(lin2b)
    full = lambda shape: pl.BlockSpec(shape, lambda: tuple(0 for _ in shape))
    out = pl.pallas_call(
        _readout_body,
        in_specs=[full((G, H))] * 3
        + [full((H, 2 * H)), full((1, 2 * H))] * 3
        + [full((2 * H, 128)), full((1, 128))],
        out_specs=full((G, 128)),
        out_shape=jax.ShapeDtypeStruct((G, 128), jnp.float32),
    )(pooled[0], pooled[1], pooled[2],
      lin1[0]["W"], lin1[0]["b"].reshape(1, 2 * H),
      lin1[1]["W"], lin1[1]["b"].reshape(1, 2 * H),
      lin1[2]["W"], lin1[2]["b"].reshape(1, 2 * H),
      w2f, b2f)
    return out[:, :C]


def _segsum_plus(x_src, src, dst, x_dst):
    return x_dst + jax.ops.segment_sum(x_src[src], dst,
                                       num_segments=x_dst.shape[0])


def kernel(x0, x1, x2, up_index0, up_index1, boundary_src1, boundary_dst1,
           boundary_src2, boundary_dst2, batch0, batch1, batch2, params):
    up0s, up0d = up_index0[0], up_index0[1]
    up1s, up1d = up_index1[0], up_index1[1]

    xs = [x0, x1, x2]
    for l in range(L):
        dims = params["layers"][l]["dims"]
        out_up0 = _segsum_plus(xs[0], up0s, up0d, xs[0])
        out_up1 = _segsum_plus(xs[1], up1s, up1d, xs[1])
        out_b1 = _segsum_plus(xs[0], boundary_src1, boundary_dst1, xs[1])
        out_b2 = _segsum_plus(xs[1], boundary_src2, boundary_dst2, xs[2])
        n0 = _tc_mlp(out_up0, xs[0], dims[0])
        n1 = _tc_mlp(out_up1, out_b1, dims[1])
        n2 = _tc_mlp(xs[2], out_b2, dims[2])
        xs = [n0, n1, n2]

    pooled = [_tc_pool(xs[d], [batch0, batch1, batch2][d]) for d in range(3)]
    return _tc_readout(pooled, params["lin1"], params["lin2W"], params["lin2b"])
